# Initial kernel scaffold; baseline (speedup 1.0000x reference)
import jax
import jax.numpy as jnp
from jax.experimental import pallas as pl


def kernel(x, w_ke, bias2d):
    raise NotImplementedError("write your pallas kernel here")


# speedup vs baseline: 1.5738x; 1.5738x over previous
import jax
import jax.numpy as jnp
from jax.experimental import pallas as pl
from jax.experimental.pallas import tpu as pltpu

_PD, _PH, _PW = 2, 16, 16


def _fused_patch_proj_kernel(x_ref, w_ref, b_ref, o_ref):
    _, c, _, pd, gh, ph, gw, pw = x_ref.shape
    k_total = c * pd * ph * pw
    xb = x_ref[...].astype(jnp.bfloat16)
    xb = xb.reshape(c * pd, gh, ph, gw, pw)
    patches = jnp.transpose(xb, (1, 3, 0, 2, 4))
    patches = patches.reshape(gh * gw, k_total)
    acc = jnp.dot(patches, w_ref[...], preferred_element_type=jnp.float32)
    o_ref[...] = (acc + b_ref[...]).reshape(o_ref.shape)


def kernel(x, w_ke, bias2d):
    n, c, d, h, w = x.shape
    e = w_ke.shape[1]
    pd, ph, pw = _PD, _PH, _PW
    gd, gh, gw = d // pd, h // ph, w // pw
    k_total = c * pd * ph * pw

    x_r = x.reshape(n, c, gd, pd, gh, ph, gw, pw)

    out = pl.pallas_call(
        _fused_patch_proj_kernel,
        out_shape=jax.ShapeDtypeStruct((n, gd, gh * gw, e), jnp.float32),
        grid=(n, gd),
        in_specs=[
            pl.BlockSpec((1, c, 1, pd, gh, ph, gw, pw),
                         lambda i, j: (i, 0, j, 0, 0, 0, 0, 0)),
            pl.BlockSpec((k_total, e), lambda i, j: (0, 0)),
            pl.BlockSpec((1, e), lambda i, j: (0, 0)),
        ],
        out_specs=pl.BlockSpec((1, 1, gh * gw, e), lambda i, j: (i, j, 0, 0)),
        compiler_params=pltpu.CompilerParams(
            dimension_semantics=("parallel", "parallel"),
        ),
        cost_estimate=pl.CostEstimate(
            flops=2 * n * gd * gh * gw * k_total * e,
            transcendentals=0,
            bytes_accessed=(n * c * d * h * w * 4 + k_total * e * 2
                            + n * gd * gh * gw * e * 4 + e * 4),
        ),
    )(x_r, w_ke, bias2d)

    return out.reshape(n, gd * gh * gw, e)


# speedup vs baseline: 2.9448x; 1.8712x over previous
import jax
import jax.numpy as jnp
from jax.experimental import pallas as pl
from jax.experimental.pallas import tpu as pltpu

_PD, _PH, _PW = 2, 16, 16


def _fused_patch_proj_kernel(x_ref, w_ref, b_ref, o_ref):
    _, c, pd, h, w = x_ref.shape
    ph, pw = _PH, _PW
    gh, gw = h // ph, w // pw
    k_total = c * pd * ph * pw
    xb = x_ref[...].astype(jnp.bfloat16)
    xb = xb.reshape(c * pd, gh, ph, gw, pw)
    patches = jnp.transpose(xb, (1, 3, 0, 2, 4))
    patches = patches.reshape(gh * gw, k_total)
    acc = jnp.dot(patches, w_ref[...], preferred_element_type=jnp.float32)
    o_ref[...] = (acc + b_ref[...]).reshape(o_ref.shape)


def kernel(x, w_ke, bias2d):
    n, c, d, h, w = x.shape
    e = w_ke.shape[1]
    pd, ph, pw = _PD, _PH, _PW
    gd, gh, gw = d // pd, h // ph, w // pw
    k_total = c * pd * ph * pw

    out = pl.pallas_call(
        _fused_patch_proj_kernel,
        out_shape=jax.ShapeDtypeStruct((n, gd, gh * gw, e), jnp.float32),
        grid=(n, gd),
        in_specs=[
            pl.BlockSpec((1, c, pd, h, w), lambda i, j: (i, 0, j, 0, 0)),
            pl.BlockSpec((k_total, e), lambda i, j: (0, 0)),
            pl.BlockSpec((1, e), lambda i, j: (0, 0)),
        ],
        out_specs=pl.BlockSpec((1, 1, gh * gw, e), lambda i, j: (i, j, 0, 0)),
        compiler_params=pltpu.CompilerParams(
            dimension_semantics=("parallel", "parallel"),
        ),
        cost_estimate=pl.CostEstimate(
            flops=2 * n * gd * gh * gw * k_total * e,
            transcendentals=0,
            bytes_accessed=(n * c * d * h * w * 4 + k_total * e * 2
                            + n * gd * gh * gw * e * 4 + e * 4),
        ),
    )(x, w_ke, bias2d)

    return out.reshape(n, gd * gh * gw, e)


# speedup vs baseline: 3.8461x; 1.3061x over previous
import jax
import jax.numpy as jnp
from jax.experimental import pallas as pl
from jax.experimental.pallas import tpu as pltpu

_PD, _PH, _PW = 2, 16, 16
_GD_BLK = 2


def _fused_patch_proj_kernel(x_ref, w_ref, b_ref, o_ref):
    _, c, d_blk, h, w = x_ref.shape
    pd, ph, pw = _PD, _PH, _PW
    gdb, gh, gw = d_blk // pd, h // ph, w // pw
    k_total = c * pd * ph * pw
    xb = x_ref[...].astype(jnp.bfloat16)
    xb = xb.reshape(c, gdb, pd, gh, ph, gw, pw)
    patches = jnp.transpose(xb, (1, 3, 5, 0, 2, 4, 6))
    patches = patches.reshape(gdb * gh * gw, k_total)
    acc = jnp.dot(patches, w_ref[...], preferred_element_type=jnp.float32)
    o_ref[...] = (acc + b_ref[...]).reshape(o_ref.shape)


def kernel(x, w_ke, bias2d):
    n, c, d, h, w = x.shape
    e = w_ke.shape[1]
    pd, ph, pw = _PD, _PH, _PW
    gd, gh, gw = d // pd, h // ph, w // pw
    k_total = c * pd * ph * pw
    m_blk = _GD_BLK * gh * gw

    out = pl.pallas_call(
        _fused_patch_proj_kernel,
        out_shape=jax.ShapeDtypeStruct((n, gd * gh * gw, e), jnp.float32),
        grid=(n, gd // _GD_BLK),
        in_specs=[
            pl.BlockSpec((1, c, _GD_BLK * pd, h, w),
                         lambda i, j: (i, 0, j, 0, 0)),
            pl.BlockSpec((k_total, e), lambda i, j: (0, 0)),
            pl.BlockSpec((1, e), lambda i, j: (0, 0)),
        ],
        out_specs=pl.BlockSpec((1, m_blk, e), lambda i, j: (i, j, 0)),
        compiler_params=pltpu.CompilerParams(
            dimension_semantics=("parallel", "parallel"),
        ),
        cost_estimate=pl.CostEstimate(
            flops=2 * n * gd * gh * gw * k_total * e,
            transcendentals=0,
            bytes_accessed=(n * c * d * h * w * 4 + k_total * e * 2
                            + n * gd * gh * gw * e * 4 + e * 4),
        ),
    )(x, w_ke, bias2d)

    return out


# speedup vs baseline: 3.8986x; 1.0137x over previous
import jax
import jax.numpy as jnp
from jax.experimental import pallas as pl
from jax.experimental.pallas import tpu as pltpu

_PD, _PH, _PW = 2, 16, 16
_GD_BLK = 4


def _fused_patch_proj_kernel(x_ref, w_ref, b_ref, o_ref):
    _, c, d_blk, h, w = x_ref.shape
    pd, ph, pw = _PD, _PH, _PW
    gdb, gh, gw = d_blk // pd, h // ph, w // pw
    k_total = c * pd * ph * pw
    xb = x_ref[...].astype(jnp.bfloat16)
    xb = xb.reshape(c, gdb, pd, gh, ph, gw, pw)
    patches = jnp.transpose(xb, (1, 3, 5, 0, 2, 4, 6))
    patches = patches.reshape(gdb * gh * gw, k_total)
    acc = jnp.dot(patches, w_ref[...], preferred_element_type=jnp.float32)
    o_ref[...] = (acc + b_ref[...]).reshape(o_ref.shape)


def kernel(x, w_ke, bias2d):
    n, c, d, h, w = x.shape
    e = w_ke.shape[1]
    pd, ph, pw = _PD, _PH, _PW
    gd, gh, gw = d // pd, h // ph, w // pw
    k_total = c * pd * ph * pw
    m_blk = _GD_BLK * gh * gw

    out = pl.pallas_call(
        _fused_patch_proj_kernel,
        out_shape=jax.ShapeDtypeStruct((n, gd * gh * gw, e), jnp.float32),
        grid=(n, gd // _GD_BLK),
        in_specs=[
            pl.BlockSpec((1, c, _GD_BLK * pd, h, w),
                         lambda i, j: (i, 0, j, 0, 0)),
            pl.BlockSpec((k_total, e), lambda i, j: (0, 0)),
            pl.BlockSpec((1, e), lambda i, j: (0, 0)),
        ],
        out_specs=pl.BlockSpec((1, m_blk, e), lambda i, j: (i, j, 0)),
        compiler_params=pltpu.CompilerParams(
            dimension_semantics=("parallel", "parallel"),
        ),
        cost_estimate=pl.CostEstimate(
            flops=2 * n * gd * gh * gw * k_total * e,
            transcendentals=0,
            bytes_accessed=(n * c * d * h * w * 4 + k_total * e * 2
                            + n * gd * gh * gw * e * 4 + e * 4),
        ),
    )(x, w_ke, bias2d)

    return out
